# baseline (speedup 1.0000x reference)
import functools

import jax
import jax.numpy as jnp
from jax import lax
from jax.experimental import pallas as pl
from jax.experimental.pallas import tpu as pltpu
from jax.experimental.pallas import tpu_sc as plsc

_BR = 1000


def _dense(x, wt, b):
    N, D = x.shape

    def body(x_ref, w_ref, b_ref, o_ref):
        o_ref[...] = (
            jnp.dot(x_ref[...], w_ref[...], preferred_element_type=jnp.float32)
            + b_ref[...]
        )

    return pl.pallas_call(
        body,
        grid=(N // _BR,),
        in_specs=[
            pl.BlockSpec((_BR, D), lambda i: (i, 0)),
            pl.BlockSpec((D, D), lambda i: (0, 0)),
            pl.BlockSpec((1, D), lambda i: (0, 0)),
        ],
        out_specs=pl.BlockSpec((_BR, D), lambda i: (i, 0)),
        out_shape=jax.ShapeDtypeStruct((N, D), jnp.float32),
    )(x, wt, b)


def _combine_relu_dense(t, p, wt, b):
    N, D = t.shape
    NC = p.shape[0]

    def body(t_ref, p_ref, w_ref, b_ref, o_ref):
        h = t_ref[...] + jnp.sum(p_ref[...], axis=0)
        h = jnp.maximum(h, 0.0)
        o_ref[...] = (
            jnp.dot(h, w_ref[...], preferred_element_type=jnp.float32) + b_ref[...]
        )

    return pl.pallas_call(
        body,
        grid=(N // _BR,),
        in_specs=[
            pl.BlockSpec((_BR, D), lambda i: (i, 0)),
            pl.BlockSpec((NC, _BR, D), lambda i: (0, i, 0)),
            pl.BlockSpec((D, D), lambda i: (0, 0)),
            pl.BlockSpec((1, D), lambda i: (0, 0)),
        ],
        out_specs=pl.BlockSpec((_BR, D), lambda i: (i, 0)),
        out_shape=jax.ShapeDtypeStruct((N, D), jnp.float32),
    )(t, p, wt, b)


def _combine(t, p):
    N, D = t.shape
    NC = p.shape[0]

    def body(t_ref, p_ref, o_ref):
        o_ref[...] = t_ref[...] + jnp.sum(p_ref[...], axis=0)

    return pl.pallas_call(
        body,
        grid=(N // _BR,),
        in_specs=[
            pl.BlockSpec((_BR, D), lambda i: (i, 0)),
            pl.BlockSpec((NC, _BR, D), lambda i: (0, i, 0)),
        ],
        out_specs=pl.BlockSpec((_BR, D), lambda i: (i, 0)),
        out_shape=jax.ShapeDtypeStruct((N, D), jnp.float32),
    )(t, p)


def _sc_aggregate(t, row3, col3, zeros):
    N, D = t.shape
    NW, nch, CH = row3.shape
    info = plsc.get_sparse_core_info()
    NC, NS = info.num_cores, info.num_subcores
    assert NW == NC * NS and N % NS == 0 and D % 16 == 0 and CH == 128
    SB = max(s for s in range(2, 17, 2) if nch % s == 0)
    NSB = nch // SB
    RPT = N // NS
    mesh = plsc.VectorSubcoreMesh(core_axis_name="c", subcore_axis_name="s")

    @functools.partial(
        pl.kernel,
        out_type=jax.ShapeDtypeStruct((NC, NS, RPT, D), jnp.float32),
        mesh=mesh,
        scratch_types=[
            pltpu.VMEM((SB, CH), jnp.int32),
            pltpu.VMEM((SB, CH), jnp.int32),
            pltpu.VMEM((CH, D), jnp.float32),
            pltpu.VMEM((CH, D), jnp.float32),
            pltpu.VMEM_SHARED((N + 8, D), jnp.float32),
            pltpu.SemaphoreType.DMA,
        ],
    )
    def k(t_hbm, row_hbm, col_hbm, z_hbm, out_hbm, rowb, colb, r0, r1, acc,
          sem):
        cid = lax.axis_index("c")
        sid = lax.axis_index("s")
        wid = sid * NC + cid
        rows = (r0, r1)

        pltpu.sync_copy(z_hbm, acc.at[pl.ds(sid * RPT, RPT)])
        plsc.subcore_barrier()

        def scat(j, b):
            pltpu.sync_copy(rows[b], acc.at[colb.at[j]], add=True)

        def sblock(s, c):
            pltpu.sync_copy(row_hbm.at[wid, pl.ds(s * SB, SB)], rowb)
            pltpu.sync_copy(col_hbm.at[wid, pl.ds(s * SB, SB)], colb)

            def body(i, c2):
                j = 2 * i
                cps = [pltpu.async_copy(t_hbm.at[rowb.at[j + b]], rows[b], sem)
                       for b in range(2)]
                for b in range(2):
                    cps[b].wait()
                    scat(j + b, b)
                return c2

            lax.fori_loop(0, SB // 2, body, 0)
            return c

        lax.fori_loop(0, NSB, sblock, 0)
        plsc.subcore_barrier()
        pltpu.sync_copy(acc.at[pl.ds(sid * RPT, RPT)], out_hbm.at[cid, sid])

    return k(t, row3, col3, zeros).reshape(NC, N, D)


def kernel(node_features, edge_index, weight0, bias0, weight1, bias1, hidden_dim):
    del hidden_dim
    N = node_features.shape[0]
    E = edge_index.shape[1]
    info = plsc.get_sparse_core_info()
    NW = info.num_cores * info.num_subcores
    EPW = E // NW
    assert E % NW == 0
    PADW = -(-EPW // 256) * 256
    row2 = edge_index[0].reshape(NW, EPW)
    col2 = edge_index[1].reshape(NW, EPW)
    pad = PADW - EPW
    row3 = jnp.pad(row2, ((0, 0), (0, pad))).reshape(NW, PADW // 128, 128)
    col3 = jnp.pad(col2, ((0, 0), (0, pad)), constant_values=N).reshape(
        NW, PADW // 128, 128)
    wt0 = jnp.transpose(weight0[0]) * 0.5
    b0 = bias0 * 0.5
    wt1 = jnp.transpose(weight1[0]) * 0.5
    b1 = bias1 * 0.5
    NS = info.num_subcores
    zeros = jnp.zeros((N // NS, node_features.shape[1]), jnp.float32)
    t0 = _dense(node_features, wt0, b0)
    p0 = _sc_aggregate(t0, row3, col3, zeros)
    t1 = _combine_relu_dense(t0, p0, wt1, b1)
    p1 = _sc_aggregate(t1, row3, col3, zeros)
    return _combine(t1, p1)


# speedup vs baseline: 1.8690x; 1.8690x over previous
import functools

import jax
import jax.numpy as jnp
from jax import lax
from jax.experimental import pallas as pl
from jax.experimental.pallas import tpu as pltpu
from jax.experimental.pallas import tpu_sc as plsc

_BR = 1000


def _dense(x, wt, b):
    N, D = x.shape

    def body(x_ref, w_ref, b_ref, o_ref):
        o_ref[...] = (
            jnp.dot(x_ref[...], w_ref[...], preferred_element_type=jnp.float32)
            + b_ref[...]
        )

    return pl.pallas_call(
        body,
        grid=(N // _BR,),
        in_specs=[
            pl.BlockSpec((_BR, D), lambda i: (i, 0)),
            pl.BlockSpec((D, D), lambda i: (0, 0)),
            pl.BlockSpec((1, D), lambda i: (0, 0)),
        ],
        out_specs=pl.BlockSpec((_BR, D), lambda i: (i, 0)),
        out_shape=jax.ShapeDtypeStruct((N, D), jnp.float32),
    )(x, wt, b)


def _combine_relu_dense(t, p, wt, b):
    N, D = t.shape
    NC = p.shape[0]

    def body(t_ref, p_ref, w_ref, b_ref, o_ref):
        h = t_ref[...] + jnp.sum(p_ref[...], axis=0)
        h = jnp.maximum(h, 0.0)
        o_ref[...] = (
            jnp.dot(h, w_ref[...], preferred_element_type=jnp.float32) + b_ref[...]
        )

    return pl.pallas_call(
        body,
        grid=(N // _BR,),
        in_specs=[
            pl.BlockSpec((_BR, D), lambda i: (i, 0)),
            pl.BlockSpec((NC, _BR, D), lambda i: (0, i, 0)),
            pl.BlockSpec((D, D), lambda i: (0, 0)),
            pl.BlockSpec((1, D), lambda i: (0, 0)),
        ],
        out_specs=pl.BlockSpec((_BR, D), lambda i: (i, 0)),
        out_shape=jax.ShapeDtypeStruct((N, D), jnp.float32),
    )(t, p, wt, b)


def _combine(t, p):
    N, D = t.shape
    NC = p.shape[0]

    def body(t_ref, p_ref, o_ref):
        o_ref[...] = t_ref[...] + jnp.sum(p_ref[...], axis=0)

    return pl.pallas_call(
        body,
        grid=(N // _BR,),
        in_specs=[
            pl.BlockSpec((_BR, D), lambda i: (i, 0)),
            pl.BlockSpec((NC, _BR, D), lambda i: (0, i, 0)),
        ],
        out_specs=pl.BlockSpec((_BR, D), lambda i: (i, 0)),
        out_shape=jax.ShapeDtypeStruct((N, D), jnp.float32),
    )(t, p)


def _sc_aggregate(t, row3, col3, zeros):
    N, D = t.shape
    NW, nch, CH = row3.shape
    info = plsc.get_sparse_core_info()
    NC, NS = info.num_cores, info.num_subcores
    assert NW == NC * NS and N % NS == 0 and D % 16 == 0
    RPT = N // NS
    mesh = plsc.VectorSubcoreMesh(core_axis_name="c", subcore_axis_name="s")

    @functools.partial(
        pl.kernel,
        out_type=jax.ShapeDtypeStruct((NC, NS, RPT, D), jnp.float32),
        mesh=mesh,
        scratch_types=[
            pltpu.VMEM((nch, CH), jnp.int32),
            pltpu.VMEM((nch, CH), jnp.int32),
            pltpu.VMEM((CH, D), jnp.float32),
            pltpu.VMEM_SHARED((N, D), jnp.float32),
            pltpu.SemaphoreType.DMA,
        ],
    )
    def k(t_hbm, row_hbm, col_hbm, z_hbm, out_hbm, rowb, colb, rows, acc,
          sem):
        cid = lax.axis_index("c")
        sid = lax.axis_index("s")
        wid = sid * NC + cid

        pltpu.sync_copy(z_hbm, acc.at[pl.ds(sid * RPT, RPT)])
        pltpu.sync_copy(row_hbm.at[wid], rowb)
        pltpu.sync_copy(col_hbm.at[wid], colb)
        plsc.subcore_barrier()

        def body(j, c):
            pltpu.async_copy(t_hbm.at[rowb.at[j]], rows, sem).wait()
            pltpu.sync_copy(rows, acc.at[colb.at[j]], add=True)
            return c

        lax.fori_loop(0, nch, body, 0)
        plsc.subcore_barrier()
        pltpu.sync_copy(acc.at[pl.ds(sid * RPT, RPT)], out_hbm.at[cid, sid])

    return k(t, row3, col3, zeros).reshape(NC, N, D)


def kernel(node_features, edge_index, weight0, bias0, weight1, bias1, hidden_dim):
    del hidden_dim
    N = node_features.shape[0]
    E = edge_index.shape[1]
    info = plsc.get_sparse_core_info()
    NW = info.num_cores * info.num_subcores
    EPW = E // NW
    assert E % NW == 0
    CH = max(c for c in range(8, 129, 8) if EPW % c == 0)
    row3 = edge_index[0].reshape(NW, EPW // CH, CH)
    col3 = edge_index[1].reshape(NW, EPW // CH, CH)
    wt0 = jnp.transpose(weight0[0]) * 0.5
    b0 = bias0 * 0.5
    wt1 = jnp.transpose(weight1[0]) * 0.5
    b1 = bias1 * 0.5
    NS = info.num_subcores
    zeros = jnp.zeros((N // NS, node_features.shape[1]), jnp.float32)
    t0 = _dense(node_features, wt0, b0)
    p0 = _sc_aggregate(t0, row3, col3, zeros)
    t1 = _combine_relu_dense(t0, p0, wt1, b1)
    p1 = _sc_aggregate(t1, row3, col3, zeros)
    return _combine(t1, p1)


# speedup vs baseline: 2.3186x; 1.2405x over previous
import functools

import jax
import jax.numpy as jnp
from jax import lax
from jax.experimental import pallas as pl
from jax.experimental.pallas import tpu as pltpu
from jax.experimental.pallas import tpu_sc as plsc

_BR = 1000


def _dense(x, wt, b):
    N, D = x.shape

    def body(x_ref, w_ref, b_ref, o_ref):
        o_ref[...] = (
            jnp.dot(x_ref[...], w_ref[...], preferred_element_type=jnp.float32)
            + b_ref[...]
        )

    return pl.pallas_call(
        body,
        grid=(N // _BR,),
        in_specs=[
            pl.BlockSpec((_BR, D), lambda i: (i, 0)),
            pl.BlockSpec((D, D), lambda i: (0, 0)),
            pl.BlockSpec((1, D), lambda i: (0, 0)),
        ],
        out_specs=pl.BlockSpec((_BR, D), lambda i: (i, 0)),
        out_shape=jax.ShapeDtypeStruct((N, D), jnp.float32),
    )(x, wt, b)


def _combine_relu_dense(t, p, wt, b):
    N, D = t.shape
    NC = p.shape[0]

    def body(t_ref, p_ref, w_ref, b_ref, o_ref):
        h = t_ref[...] + jnp.sum(p_ref[...], axis=0)
        h = jnp.maximum(h, 0.0)
        o_ref[...] = (
            jnp.dot(h, w_ref[...], preferred_element_type=jnp.float32) + b_ref[...]
        )

    return pl.pallas_call(
        body,
        grid=(N // _BR,),
        in_specs=[
            pl.BlockSpec((_BR, D), lambda i: (i, 0)),
            pl.BlockSpec((NC, _BR, D), lambda i: (0, i, 0)),
            pl.BlockSpec((D, D), lambda i: (0, 0)),
            pl.BlockSpec((1, D), lambda i: (0, 0)),
        ],
        out_specs=pl.BlockSpec((_BR, D), lambda i: (i, 0)),
        out_shape=jax.ShapeDtypeStruct((N, D), jnp.float32),
    )(t, p, wt, b)


def _combine(t, p):
    N, D = t.shape
    NC = p.shape[0]

    def body(t_ref, p_ref, o_ref):
        o_ref[...] = t_ref[...] + jnp.sum(p_ref[...], axis=0)

    return pl.pallas_call(
        body,
        grid=(N // _BR,),
        in_specs=[
            pl.BlockSpec((_BR, D), lambda i: (i, 0)),
            pl.BlockSpec((NC, _BR, D), lambda i: (0, i, 0)),
        ],
        out_specs=pl.BlockSpec((_BR, D), lambda i: (i, 0)),
        out_shape=jax.ShapeDtypeStruct((N, D), jnp.float32),
    )(t, p)


def _sc_aggregate(t, row3, col3, zeros):
    N, D = t.shape
    NW, nch, CH = row3.shape
    info = plsc.get_sparse_core_info()
    NC, NS = info.num_cores, info.num_subcores
    assert NW == NC * NS and N % NS == 0 and D % 16 == 0
    RPT = N // NS
    SB = max(s for s in range(2, 17, 2) if nch % s == 0)
    NSB = nch // SB
    mesh = plsc.VectorSubcoreMesh(core_axis_name="c", subcore_axis_name="s")

    @functools.partial(
        pl.kernel,
        out_type=jax.ShapeDtypeStruct((NC, NS, RPT, D), jnp.float32),
        mesh=mesh,
        scratch_types=[
            pltpu.VMEM((SB, CH), jnp.int32),
            pltpu.VMEM((SB, CH), jnp.int32),
            pltpu.VMEM((CH, D), jnp.float32),
            pltpu.VMEM((CH, D), jnp.float32),
            pltpu.VMEM_SHARED((N, D), jnp.float32),
            pltpu.SemaphoreType.DMA,
            pltpu.SemaphoreType.DMA,
        ],
    )
    def k(t_hbm, row_hbm, col_hbm, z_hbm, out_hbm, rowb, colb, r0, r1, acc,
          s0, s1):
        cid = lax.axis_index("c")
        sid = lax.axis_index("s")
        wid = sid * NC + cid
        rows = (r0, r1)
        sems = (s0, s1)

        pltpu.sync_copy(z_hbm, acc.at[pl.ds(sid * RPT, RPT)])
        plsc.subcore_barrier()

        def sblock(s, c):
            pltpu.sync_copy(row_hbm.at[wid, pl.ds(s * SB, SB)], rowb)
            pltpu.sync_copy(col_hbm.at[wid, pl.ds(s * SB, SB)], colb)

            def body(i, c2):
                j = 2 * i
                cps = [pltpu.async_copy(t_hbm.at[rowb.at[j + b]], rows[b],
                                        sems[b]) for b in range(2)]
                for b in range(2):
                    cps[b].wait()
                    pltpu.sync_copy(rows[b], acc.at[colb.at[j + b]], add=True)
                return c2

            lax.fori_loop(0, SB // 2, body, 0)
            return c

        lax.fori_loop(0, NSB, sblock, 0)
        plsc.subcore_barrier()
        pltpu.sync_copy(acc.at[pl.ds(sid * RPT, RPT)], out_hbm.at[cid, sid])

    return k(t, row3, col3, zeros).reshape(NC, N, D)


def kernel(node_features, edge_index, weight0, bias0, weight1, bias1, hidden_dim):
    del hidden_dim
    N = node_features.shape[0]
    E = edge_index.shape[1]
    info = plsc.get_sparse_core_info()
    NW = info.num_cores * info.num_subcores
    EPW = E // NW
    assert E % NW == 0
    CH = max(c for c in range(1, 129) if EPW % c == 0 and (EPW // c) % 2 == 0)
    row3 = edge_index[0].reshape(NW, EPW // CH, CH)
    col3 = edge_index[1].reshape(NW, EPW // CH, CH)
    wt0 = jnp.transpose(weight0[0]) * 0.5
    b0 = bias0 * 0.5
    wt1 = jnp.transpose(weight1[0]) * 0.5
    b1 = bias1 * 0.5
    NS = info.num_subcores
    zeros = jnp.zeros((N // NS, node_features.shape[1]), jnp.float32)
    t0 = _dense(node_features, wt0, b0)
    p0 = _sc_aggregate(t0, row3, col3, zeros)
    t1 = _combine_relu_dense(t0, p0, wt1, b1)
    p1 = _sc_aggregate(t1, row3, col3, zeros)
    return _combine(t1, p1)


# speedup vs baseline: 2.8465x; 1.2277x over previous
import functools

import jax
import jax.numpy as jnp
from jax import lax
from jax.experimental import pallas as pl
from jax.experimental.pallas import tpu as pltpu
from jax.experimental.pallas import tpu_sc as plsc

_BR = 1000


def _dense(x, wt, b):
    N, D = x.shape

    def body(x_ref, w_ref, b_ref, o_ref):
        o_ref[...] = (
            jnp.dot(x_ref[...], w_ref[...], preferred_element_type=jnp.float32)
            + b_ref[...]
        )

    return pl.pallas_call(
        body,
        grid=(N // _BR,),
        in_specs=[
            pl.BlockSpec((_BR, D), lambda i: (i, 0)),
            pl.BlockSpec((D, D), lambda i: (0, 0)),
            pl.BlockSpec((1, D), lambda i: (0, 0)),
        ],
        out_specs=pl.BlockSpec((_BR, D), lambda i: (i, 0)),
        out_shape=jax.ShapeDtypeStruct((N, D), jnp.float32),
    )(x, wt, b)


def _combine_relu_dense(t, p, wt, b):
    N, D = t.shape
    NC = p.shape[0]

    def body(t_ref, p_ref, w_ref, b_ref, o_ref):
        h = t_ref[...] + jnp.sum(p_ref[...], axis=0)
        h = jnp.maximum(h, 0.0)
        o_ref[...] = (
            jnp.dot(h, w_ref[...], preferred_element_type=jnp.float32) + b_ref[...]
        )

    return pl.pallas_call(
        body,
        grid=(N // _BR,),
        in_specs=[
            pl.BlockSpec((_BR, D), lambda i: (i, 0)),
            pl.BlockSpec((NC, _BR, D), lambda i: (0, i, 0)),
            pl.BlockSpec((D, D), lambda i: (0, 0)),
            pl.BlockSpec((1, D), lambda i: (0, 0)),
        ],
        out_specs=pl.BlockSpec((_BR, D), lambda i: (i, 0)),
        out_shape=jax.ShapeDtypeStruct((N, D), jnp.float32),
    )(t, p, wt, b)


def _combine(t, p):
    N, D = t.shape
    NC = p.shape[0]

    def body(t_ref, p_ref, o_ref):
        o_ref[...] = t_ref[...] + jnp.sum(p_ref[...], axis=0)

    return pl.pallas_call(
        body,
        grid=(N // _BR,),
        in_specs=[
            pl.BlockSpec((_BR, D), lambda i: (i, 0)),
            pl.BlockSpec((NC, _BR, D), lambda i: (0, i, 0)),
        ],
        out_specs=pl.BlockSpec((_BR, D), lambda i: (i, 0)),
        out_shape=jax.ShapeDtypeStruct((N, D), jnp.float32),
    )(t, p)


def _sc_aggregate(t, row3, col3, zeros):
    N, D = t.shape
    NW, nch, CH = row3.shape
    info = plsc.get_sparse_core_info()
    NC, NS = info.num_cores, info.num_subcores
    assert NW == NC * NS and N % NS == 0 and D % 16 == 0
    RPT = N // NS
    SB = max(s for s in range(2, 17, 2) if nch % s == 0)
    NSB = nch // SB
    mesh = plsc.VectorSubcoreMesh(core_axis_name="c", subcore_axis_name="s")

    @functools.partial(
        pl.kernel,
        out_type=jax.ShapeDtypeStruct((NC, NS, RPT, D), jnp.float32),
        mesh=mesh,
        scratch_types=[
            pltpu.VMEM((SB, CH), jnp.int32),
            pltpu.VMEM((SB, CH), jnp.int32),
            pltpu.VMEM((CH, D), jnp.float32),
            pltpu.VMEM((CH, D), jnp.float32),
            pltpu.VMEM_SHARED((N, D), jnp.float32),
            pltpu.SemaphoreType.DMA,
            pltpu.SemaphoreType.DMA,
        ],
    )
    def k(t_hbm, row_hbm, col_hbm, z_hbm, out_hbm, rowb, colb, r0, r1, acc,
          s0, s1):
        cid = lax.axis_index("c")
        sid = lax.axis_index("s")
        wid = sid * NC + cid
        rows = (r0, r1)
        sems = (s0, s1)

        pltpu.sync_copy(z_hbm, acc.at[pl.ds(sid * RPT, RPT)])
        plsc.subcore_barrier()

        def fire(j, b):
            pltpu.async_copy(t_hbm.at[rowb.at[j]], rows[b], sems[b])

        def wait(j, b):
            pltpu.make_async_copy(t_hbm.at[rowb.at[j]], rows[b], sems[b]).wait()

        def scat(j, b):
            pltpu.sync_copy(rows[b], acc.at[colb.at[j]], add=True)

        def sblock(s, c):
            pltpu.sync_copy(row_hbm.at[wid, pl.ds(s * SB, SB)], rowb)
            pltpu.sync_copy(col_hbm.at[wid, pl.ds(s * SB, SB)], colb)
            fire(0, 0)
            fire(1, 1)

            def body(i, c2):
                j = 2 * i
                for b in range(2):
                    wait(j + b, b)
                    scat(j + b, b)
                    fire(j + b + 2, b)
                return c2

            lax.fori_loop(0, SB // 2 - 1, body, 0)
            for b in range(2):
                wait(SB - 2 + b, b)
                scat(SB - 2 + b, b)
            return c

        lax.fori_loop(0, NSB, sblock, 0)
        plsc.subcore_barrier()
        pltpu.sync_copy(acc.at[pl.ds(sid * RPT, RPT)], out_hbm.at[cid, sid])

    return k(t, row3, col3, zeros).reshape(NC, N, D)


def kernel(node_features, edge_index, weight0, bias0, weight1, bias1, hidden_dim):
    del hidden_dim
    N = node_features.shape[0]
    E = edge_index.shape[1]
    info = plsc.get_sparse_core_info()
    NW = info.num_cores * info.num_subcores
    EPW = E // NW
    assert E % NW == 0
    CH = max(c for c in range(1, 129) if EPW % c == 0 and (EPW // c) % 2 == 0)
    row3 = edge_index[0].reshape(NW, EPW // CH, CH)
    col3 = edge_index[1].reshape(NW, EPW // CH, CH)
    wt0 = jnp.transpose(weight0[0]) * 0.5
    b0 = bias0 * 0.5
    wt1 = jnp.transpose(weight1[0]) * 0.5
    b1 = bias1 * 0.5
    NS = info.num_subcores
    zeros = jnp.zeros((N // NS, node_features.shape[1]), jnp.float32)
    t0 = _dense(node_features, wt0, b0)
    p0 = _sc_aggregate(t0, row3, col3, zeros)
    t1 = _combine_relu_dense(t0, p0, wt1, b1)
    p1 = _sc_aggregate(t1, row3, col3, zeros)
    return _combine(t1, p1)


# speedup vs baseline: 3.1262x; 1.0983x over previous
import functools

import jax
import jax.numpy as jnp
from jax import lax
from jax.experimental import pallas as pl
from jax.experimental.pallas import tpu as pltpu
from jax.experimental.pallas import tpu_sc as plsc

_BR = 1000


_DN = (((1,), (1,)), ((), ()))


def _dense(x, w, b):
    N, D = x.shape

    def body(x_ref, w_ref, b_ref, o_ref):
        o_ref[...] = (
            lax.dot_general(x_ref[...], w_ref[...], _DN,
                            preferred_element_type=jnp.float32)
            + b_ref[...]
        ) * 0.5

    return pl.pallas_call(
        body,
        grid=(N // _BR,),
        in_specs=[
            pl.BlockSpec((_BR, D), lambda i: (i, 0)),
            pl.BlockSpec((D, D), lambda i: (0, 0)),
            pl.BlockSpec((1, D), lambda i: (0, 0)),
        ],
        out_specs=pl.BlockSpec((_BR, D), lambda i: (i, 0)),
        out_shape=jax.ShapeDtypeStruct((N, D), jnp.float32),
    )(x, w, b)


def _combine_relu_dense(t, p, w, b):
    N, D = t.shape
    NC = p.shape[0]

    def body(t_ref, p_ref, w_ref, b_ref, o_ref):
        h = t_ref[...] + jnp.sum(p_ref[...], axis=0)
        h = jnp.maximum(h, 0.0)
        o_ref[...] = (
            lax.dot_general(h, w_ref[...], _DN,
                            preferred_element_type=jnp.float32)
            + b_ref[...]
        ) * 0.5

    return pl.pallas_call(
        body,
        grid=(N // _BR,),
        in_specs=[
            pl.BlockSpec((_BR, D), lambda i: (i, 0)),
            pl.BlockSpec((NC, _BR, D), lambda i: (0, i, 0)),
            pl.BlockSpec((D, D), lambda i: (0, 0)),
            pl.BlockSpec((1, D), lambda i: (0, 0)),
        ],
        out_specs=pl.BlockSpec((_BR, D), lambda i: (i, 0)),
        out_shape=jax.ShapeDtypeStruct((N, D), jnp.float32),
    )(t, p, w, b)


def _combine(t, p):
    N, D = t.shape
    NC = p.shape[0]

    def body(t_ref, p_ref, o_ref):
        o_ref[...] = t_ref[...] + jnp.sum(p_ref[...], axis=0)

    return pl.pallas_call(
        body,
        grid=(N // _BR,),
        in_specs=[
            pl.BlockSpec((_BR, D), lambda i: (i, 0)),
            pl.BlockSpec((NC, _BR, D), lambda i: (0, i, 0)),
        ],
        out_specs=pl.BlockSpec((_BR, D), lambda i: (i, 0)),
        out_shape=jax.ShapeDtypeStruct((N, D), jnp.float32),
    )(t, p)


def _sc_aggregate(t, rc3, zeros):
    N, D = t.shape
    _, NW, nch, CH = rc3.shape
    info = plsc.get_sparse_core_info()
    NC, NS = info.num_cores, info.num_subcores
    assert NW == NC * NS and N % NS == 0 and D % 16 == 0
    RPT = N // NS
    SB = max(s for s in range(2, 41, 2) if nch % s == 0)
    NSB = nch // SB
    mesh = plsc.VectorSubcoreMesh(core_axis_name="c", subcore_axis_name="s")

    @functools.partial(
        pl.kernel,
        out_type=jax.ShapeDtypeStruct((NC, NS, RPT, D), jnp.float32),
        mesh=mesh,
        scratch_types=[
            pltpu.VMEM((SB, CH), jnp.int32),
            pltpu.VMEM((SB, CH), jnp.int32),
            pltpu.VMEM((CH, D), jnp.float32),
            pltpu.VMEM((CH, D), jnp.float32),
            pltpu.VMEM_SHARED((N, D), jnp.float32),
            pltpu.SemaphoreType.DMA,
            pltpu.SemaphoreType.DMA,
        ],
    )
    def k(t_hbm, rc_hbm, z_hbm, out_hbm, rowb, colb, r0, r1, acc, s0, s1):
        cid = lax.axis_index("c")
        sid = lax.axis_index("s")
        wid = sid * NC + cid
        rows = (r0, r1)
        sems = (s0, s1)

        pltpu.sync_copy(z_hbm, acc.at[pl.ds(sid * RPT, RPT)])
        plsc.subcore_barrier()

        def fire(j, b):
            pltpu.async_copy(t_hbm.at[rowb.at[j]], rows[b], sems[b])

        def wait(j, b):
            pltpu.make_async_copy(t_hbm.at[rowb.at[j]], rows[b], sems[b]).wait()

        def scat(j, b):
            pltpu.sync_copy(rows[b], acc.at[colb.at[j]], add=True)

        def sblock(s, c):
            pltpu.sync_copy(rc_hbm.at[0, wid, pl.ds(s * SB, SB)], rowb)
            pltpu.sync_copy(rc_hbm.at[1, wid, pl.ds(s * SB, SB)], colb)
            fire(0, 0)
            fire(1, 1)

            def body(i, c2):
                j = 2 * i
                for b in range(2):
                    wait(j + b, b)
                    scat(j + b, b)
                    fire(j + b + 2, b)
                return c2

            lax.fori_loop(0, SB // 2 - 1, body, 0)
            for b in range(2):
                wait(SB - 2 + b, b)
                scat(SB - 2 + b, b)
            return c

        lax.fori_loop(0, NSB, sblock, 0)
        plsc.subcore_barrier()
        pltpu.sync_copy(acc.at[pl.ds(sid * RPT, RPT)], out_hbm.at[cid, sid])

    return k(t, rc3, zeros).reshape(NC, N, D)


def kernel(node_features, edge_index, weight0, bias0, weight1, bias1, hidden_dim):
    del hidden_dim
    N = node_features.shape[0]
    E = edge_index.shape[1]
    info = plsc.get_sparse_core_info()
    NW = info.num_cores * info.num_subcores
    EPW = E // NW
    assert E % NW == 0
    CH = max(c for c in range(1, 129) if EPW % c == 0 and (EPW // c) % 2 == 0)
    rc3 = edge_index.reshape(2, NW, EPW // CH, CH)
    NS = info.num_subcores
    zeros = jnp.zeros((N // NS, node_features.shape[1]), jnp.float32)
    t0 = _dense(node_features, weight0[0], bias0)
    p0 = _sc_aggregate(t0, rc3, zeros)
    t1 = _combine_relu_dense(t0, p0, weight1[0], bias1)
    p1 = _sc_aggregate(t1, rc3, zeros)
    return _combine(t1, p1)
